# baseline (speedup 1.0000x reference)
import jax
import jax.numpy as jnp
from jax.experimental import pallas as pl
from jax.experimental.pallas import tpu as pltpu

_DOT_DIMS = (((1,), (0,)), ((), ()))
_HB = 8


def _head_kernel(x_ref, wc_ref, bc_ref, wr_ref, br_ref, wd_ref, bd_ref,
                 cls_ref, reg_ref, dir_ref):
    wc = wc_ref[...]
    wr = wr_ref[...]
    wd = wd_ref[...]
    bc = bc_ref[...]
    br = br_ref[...]
    bd = bd_ref[...]
    for h in range(_HB):
        xb = x_ref[0, :, h, :]
        cls_ref[0, :, h, :] = jax.lax.dot_general(
            wc, xb, _DOT_DIMS, preferred_element_type=jnp.float32) + bc
        reg_ref[0, :, h, :] = jax.lax.dot_general(
            wr, xb, _DOT_DIMS, preferred_element_type=jnp.float32) + br
        dir_ref[0, :, h, :] = jax.lax.dot_general(
            wd, xb, _DOT_DIMS, preferred_element_type=jnp.float32) + bd


def kernel(x, W_cls, b_cls, W_reg, b_reg, W_dir, b_dir):
    B, C, H, W = x.shape
    G = H // _HB
    oc, og, od = W_cls.shape[0], W_reg.shape[0], W_dir.shape[0]
    bc = b_cls.reshape(oc, 1)
    bg = b_reg.reshape(og, 1)
    bd = b_dir.reshape(od, 1)

    def wspec(o):
        return pl.BlockSpec((o, C), lambda b, j: (0, 0))

    def bspec(o):
        return pl.BlockSpec((o, 1), lambda b, j: (0, 0))

    def ospec(o):
        return pl.BlockSpec((1, o, _HB, W), lambda b, j: (b, 0, j, 0))

    outs = pl.pallas_call(
        _head_kernel,
        grid=(B, G),
        in_specs=[
            pl.BlockSpec((1, C, _HB, W), lambda b, j: (b, 0, j, 0)),
            wspec(oc), bspec(oc), wspec(og), bspec(og), wspec(od), bspec(od),
        ],
        out_specs=[ospec(oc), ospec(og), ospec(od)],
        out_shape=[
            jax.ShapeDtypeStruct((B, oc, H, W), x.dtype),
            jax.ShapeDtypeStruct((B, og, H, W), x.dtype),
            jax.ShapeDtypeStruct((B, od, H, W), x.dtype),
        ],
        compiler_params=pltpu.CompilerParams(
            dimension_semantics=("parallel", "parallel")),
    )(x, W_cls, bc, W_reg, bg, W_dir, bd)
    return outs
